# baseline (speedup 1.0000x reference)
import functools

import jax
import jax.numpy as jnp
from jax import lax
from jax.experimental import pallas as pl
from jax.experimental.pallas import tpu as pltpu
from jax.experimental.pallas import tpu_sc as plsc

D_MODEL = 64
NUM_CORES = 2
NUM_SUBCORES = 16
NUM_WORKERS = NUM_CORES * NUM_SUBCORES

IDX_W = 128
NI = 5
CHUNK = IDX_W * NI


@functools.lru_cache(maxsize=None)
def _build_gather(B: int, V: int, D: int):
    assert B % (NUM_WORKERS * 2 * CHUNK) == 0
    b_per_w = B // NUM_WORKERS
    n_chunks = b_per_w // CHUNK
    n_pairs = n_chunks // 2

    mesh = plsc.VectorSubcoreMesh(
        core_axis_name="c", subcore_axis_name="s")

    @functools.partial(
        pl.kernel,
        out_type=jax.ShapeDtypeStruct((B, D), jnp.float32),
        mesh=mesh,
        compiler_params=pltpu.CompilerParams(use_tc_tiling_on_sc=False),
        scratch_types=[
            pltpu.VMEM((2, CHUNK), jnp.int32),
            pltpu.VMEM((2, CHUNK, D), jnp.float32),
            pltpu.SemaphoreType.DMA,
            pltpu.SemaphoreType.DMA,
            pltpu.SemaphoreType.DMA,
            pltpu.SemaphoreType.DMA,
        ],
    )
    def grab(idx_hbm, table_hbm, out_hbm, idx_v, rows_v,
             gsem0, gsem1, osem0, osem1):
        gsems = (gsem0, gsem1)
        osems = (osem0, osem1)
        wid = lax.axis_index("s") * NUM_CORES + lax.axis_index("c")
        w_base = wid * b_per_w

        def issue(c, b):
            pltpu.sync_copy(idx_hbm.at[pl.ds(w_base + c * CHUNK, CHUNK)],
                            idx_v.at[b])
            for j in range(NI):
                pltpu.async_copy(
                    table_hbm.at[idx_v.at[b, pl.ds(j * IDX_W, IDX_W)]],
                    rows_v.at[b, pl.ds(j * IDX_W, IDX_W)],
                    gsems[b])

        def drain_gather(b):
            pltpu.make_async_copy(
                table_hbm.at[pl.ds(0, CHUNK)], rows_v.at[b],
                gsems[b]).wait()

        def write(c, b):
            pltpu.async_copy(rows_v.at[b],
                             out_hbm.at[pl.ds(w_base + c * CHUNK, CHUNK)],
                             osems[b])

        def wait_write(b):
            pltpu.make_async_copy(rows_v.at[b],
                                  out_hbm.at[pl.ds(0, CHUNK)],
                                  osems[b]).wait()

        issue(0, 0)
        issue(1, 1)
        drain_gather(0)
        write(0, 0)

        @pl.loop(1, n_pairs)
        def _pair(t):
            c0 = 2 * t
            wait_write(0)
            issue(c0, 0)
            drain_gather(1)
            write(c0 - 1, 1)
            wait_write(1)
            issue(c0 + 1, 1)
            drain_gather(0)
            write(c0, 0)

        drain_gather(1)
        write(n_chunks - 1, 1)
        wait_write(0)
        wait_write(1)

    return grab


def kernel(tokens, W_E):
    B = tokens.shape[0] * tokens.shape[1]
    V, D = W_E.shape
    idx = tokens.astype(jnp.int32).reshape(B)
    out = _build_gather(B, V, D)(idx, W_E)
    return out.reshape(tokens.shape + (D,))
